# Initial kernel scaffold; baseline (speedup 1.0000x reference)
import jax
import jax.numpy as jnp
from jax.experimental import pallas as pl


def kernel(boxes, scores):
    raise NotImplementedError("write your pallas kernel here")


# speedup vs baseline: 14.4779x; 14.4779x over previous
import jax
import jax.numpy as jnp
from jax.experimental import pallas as pl
from jax.experimental.pallas import tpu as pltpu

_N = 5000
_B = 512
_NPAD = 5120
_NB = _NPAD // _B
_IOU_T = 0.9


def _nms_kernel(bs_ref, bst_ref, keep_ref, cd_ref):
    lanes = jax.lax.broadcasted_iota(jnp.int32, (1, _B), 1)

    def tile_cond(bj, bi):
        r = bs_ref[pl.ds(bj * _B, _B), :]
        x1r, y1r, x2r, y2r = r[:, 0:1], r[:, 1:2], r[:, 2:3], r[:, 3:4]
        c = bst_ref[:, pl.ds(bi * _B, _B)]
        x1c, y1c, x2c, y2c = c[0:1, :], c[1:2, :], c[2:3, :], c[3:4, :]
        ar = (x2r - x1r) * (y2r - y1r)
        ac = (x2c - x1c) * (y2c - y1c)
        xA = jnp.maximum(x1r, x1c)
        yA = jnp.maximum(y1r, y1c)
        xB = jnp.minimum(x2r, x2c)
        yB = jnp.minimum(y2r, y2c)
        inter = jnp.maximum(xB - xA, 0.0) * jnp.maximum(yB - yA, 0.0)
        union = ar + ac - inter
        iou = jnp.where(inter > 0.0, inter / jnp.maximum(union, 1e-12), 0.0)
        inside = ((x1c >= x1r) & (y1c >= y1r) & (x2c <= x2r) & (y2c <= y2r))
        return (inside | (iou > _IOU_T)).astype(jnp.float32)

    keep_blocks = []
    for bi in range(_NB):
        sup = jnp.zeros((1, _B), jnp.float32)
        for bj in range(bi):
            ct = tile_cond(bj, bi)
            sup = sup + jax.lax.dot_general(
                keep_blocks[bj], ct, (((1,), (0,)), ((), ())),
                preferred_element_type=jnp.float32)
        keep0 = (sup == 0.0).astype(jnp.float32)

        cd_ref[:, :] = tile_cond(bi, bi)

        def body(i, keep):
            row = cd_ref[pl.ds(i, 1), :]
            ki = jnp.sum(keep * (lanes == i).astype(jnp.float32))
            sup_i = ki * row * (lanes > i).astype(jnp.float32)
            return keep * (1.0 - sup_i)

        kb = jax.lax.fori_loop(0, _B, body, keep0)
        keep_blocks.append(kb)
        keep_ref[:, pl.ds(bi * _B, _B)] = kb


def _run_nms(bsp, bst):
    return pl.pallas_call(
        _nms_kernel,
        out_shape=jax.ShapeDtypeStruct((1, _NPAD), jnp.float32),
        scratch_shapes=[pltpu.VMEM((_B, _B), jnp.float32)],
    )(bsp, bst)


def kernel(boxes, scores):
    order = jnp.argsort(-scores)
    bs = jnp.take(boxes, order, axis=0)
    ss = jnp.take(scores, order)
    bsp = jnp.concatenate(
        [bs, jnp.zeros((_NPAD - _N, 4), jnp.float32)], axis=0)
    keepf = _run_nms(bsp, bsp.T)[0, :_N]
    keep = keepf > 0.5
    boxes_out = jnp.where(keep[:, None], bs, 0.0)
    scores_out = jnp.where(keep, ss, 0.0)
    return boxes_out, scores_out, keep


# speedup vs baseline: 35.7698x; 2.4706x over previous
import jax
import jax.numpy as jnp
from jax.experimental import pallas as pl
from jax.experimental.pallas import tpu as pltpu

_N = 5000
_B = 512
_NPAD = 5120
_NB = _NPAD // _B
_R = 16
_IOU_T = 0.9


def _nms_kernel(bs_ref, bst_ref, keep_ref, cd_ref):
    lanes = jax.lax.broadcasted_iota(jnp.int32, (1, _B), 1)

    def tile_cond(bj, bi):
        r = bs_ref[pl.ds(bj * _B, _B), :]
        x1r, y1r, x2r, y2r = r[:, 0:1], r[:, 1:2], r[:, 2:3], r[:, 3:4]
        c = bst_ref[:, pl.ds(bi * _B, _B)]
        x1c, y1c, x2c, y2c = c[0:1, :], c[1:2, :], c[2:3, :], c[3:4, :]
        ar = (x2r - x1r) * (y2r - y1r)
        ac = (x2c - x1c) * (y2c - y1c)
        xA = jnp.maximum(x1r, x1c)
        yA = jnp.maximum(y1r, y1c)
        xB = jnp.minimum(x2r, x2c)
        yB = jnp.minimum(y2r, y2c)
        inter = jnp.maximum(xB - xA, 0.0) * jnp.maximum(yB - yA, 0.0)
        union = ar + ac - inter
        iou = jnp.where(inter > 0.0, inter / jnp.maximum(union, 1e-12), 0.0)
        inside = ((x1c >= x1r) & (y1c >= y1r) & (x2c <= x2r) & (y2c <= y2r))
        return (inside | (iou > _IOU_T)).astype(jnp.float32)

    triu = (jax.lax.broadcasted_iota(jnp.int32, (_R, _R), 1) >
            jax.lax.broadcasted_iota(jnp.int32, (_R, _R), 0)
            ).astype(jnp.float32)
    riota = jax.lax.broadcasted_iota(jnp.int32, (_R, 1), 0)

    keep_blocks = []
    for bi in range(_NB):
        sup = jnp.zeros((1, _B), jnp.float32)
        for bj in range(bi):
            ct = tile_cond(bj, bi)
            sup = sup + jax.lax.dot_general(
                keep_blocks[bj], ct, (((1,), (0,)), ((), ())),
                preferred_element_type=jnp.float32)
        keep = (sup == 0.0).astype(jnp.float32)

        cd_ref[:, :] = tile_cond(bi, bi)

        for s in range(_B // _R):
            a = s * _R
            c_sub = cd_ref[pl.ds(a, _R), pl.ds(a, _R)]
            dirty = jnp.sum(c_sub * triu) > 0.0
            gt = (lanes > (riota + a)).astype(jnp.float32)
            crm = cd_ref[pl.ds(a, _R), :] * gt

            def clean_fn(k, a=a, crm=crm):
                ks = k[:, a:a + _R]
                sup_s = jax.lax.dot_general(
                    ks, crm, (((1,), (0,)), ((), ())),
                    preferred_element_type=jnp.float32)
                return k * (sup_s == 0.0).astype(jnp.float32)

            def dirty_fn(k, a=a):
                def body(r, kk):
                    i = a + r
                    ki = jnp.sum(kk * (lanes == i).astype(jnp.float32))
                    row = cd_ref[pl.ds(i, 1), :] * (lanes > i).astype(
                        jnp.float32)
                    return kk * (1.0 - ki * row)
                return jax.lax.fori_loop(0, _R, body, k)

            keep = jax.lax.cond(dirty, dirty_fn, clean_fn, keep)

        keep_blocks.append(keep)
        keep_ref[:, pl.ds(bi * _B, _B)] = keep


def _run_nms(bsp, bst):
    return pl.pallas_call(
        _nms_kernel,
        out_shape=jax.ShapeDtypeStruct((1, _NPAD), jnp.float32),
        scratch_shapes=[pltpu.VMEM((_B, _B), jnp.float32)],
    )(bsp, bst)


def kernel(boxes, scores):
    order = jnp.argsort(-scores)
    bs = jnp.take(boxes, order, axis=0)
    ss = jnp.take(scores, order)
    bsp = jnp.concatenate(
        [bs, jnp.zeros((_NPAD - _N, 4), jnp.float32)], axis=0)
    keepf = _run_nms(bsp, bsp.T)[0, :_N]
    keep = keepf > 0.5
    boxes_out = jnp.where(keep[:, None], bs, 0.0)
    scores_out = jnp.where(keep, ss, 0.0)
    return boxes_out, scores_out, keep


# speedup vs baseline: 44.0513x; 1.2315x over previous
import jax
import jax.numpy as jnp
from jax.experimental import pallas as pl
from jax.experimental.pallas import tpu as pltpu

_N = 5000
_B = 512
_NPAD = 5120
_NB = _NPAD // _B
_R = 16
_IOU_T = 0.9


def _nms_kernel(bs_ref, bst_ref, keep_ref, cd_ref):
    lanes = jax.lax.broadcasted_iota(jnp.int32, (1, _B), 1)

    def tile_cond(bj, bi):
        r = bs_ref[pl.ds(bj * _B, _B), :]
        x1r, y1r, x2r, y2r = r[:, 0:1], r[:, 1:2], r[:, 2:3], r[:, 3:4]
        c = bst_ref[:, pl.ds(bi * _B, _B)]
        x1c, y1c, x2c, y2c = c[0:1, :], c[1:2, :], c[2:3, :], c[3:4, :]
        ar = (x2r - x1r) * (y2r - y1r)
        ac = (x2c - x1c) * (y2c - y1c)
        xA = jnp.maximum(x1r, x1c)
        yA = jnp.maximum(y1r, y1c)
        xB = jnp.minimum(x2r, x2c)
        yB = jnp.minimum(y2r, y2c)
        inter = jnp.maximum(xB - xA, 0.0) * jnp.maximum(yB - yA, 0.0)
        union = ar + ac - inter
        iou = jnp.where(inter > 0.0, inter / jnp.maximum(union, 1e-12), 0.0)
        inside = ((x1c >= x1r) & (y1c >= y1r) & (x2c <= x2r) & (y2c <= y2r))
        return (inside | (iou > _IOU_T)).astype(jnp.float32)

    triu = (jax.lax.broadcasted_iota(jnp.int32, (_R, _R), 1) >
            jax.lax.broadcasted_iota(jnp.int32, (_R, _R), 0)
            ).astype(jnp.float32)
    riota = jax.lax.broadcasted_iota(jnp.int32, (_R, 1), 0)
    lanes16 = jax.lax.broadcasted_iota(jnp.int32, (1, _R), 1)
    gt16 = [(lanes16 > r).astype(jnp.float32) for r in range(_R)]

    keep_blocks = []
    for bi in range(_NB):
        sup = jnp.zeros((1, _B), jnp.float32)
        for bj in range(bi):
            ct = tile_cond(bj, bi)
            sup = sup + jax.lax.dot_general(
                keep_blocks[bj], ct, (((1,), (0,)), ((), ())),
                preferred_element_type=jnp.float32)
        keep = (sup == 0.0).astype(jnp.float32)

        cd_ref[:, :] = tile_cond(bi, bi)

        for s in range(_B // _R):
            a = s * _R
            c_sub = cd_ref[pl.ds(a, _R), pl.ds(a, _R)]
            dirty = jnp.sum(c_sub * triu) > 0.0
            gt = (lanes > (riota + a)).astype(jnp.float32)
            crm = cd_ref[pl.ds(a, _R), :] * gt
            ks_raw = keep[:, a:a + _R]

            def solve_fn(ks, c_sub=c_sub):
                for r in range(_R - 1):
                    ki = ks[:, r:r + 1]
                    crow = c_sub[r:r + 1, :] * gt16[r]
                    ks = ks * (1.0 - ki * crow)
                return ks

            ks = jax.lax.cond(dirty, solve_fn, lambda ks: ks, ks_raw)
            sup_s = jax.lax.dot_general(
                ks, crm, (((1,), (0,)), ((), ())),
                preferred_element_type=jnp.float32)
            keep = keep * (sup_s == 0.0).astype(jnp.float32)

        keep_blocks.append(keep)
        keep_ref[:, pl.ds(bi * _B, _B)] = keep


def _run_nms(bsp, bst):
    return pl.pallas_call(
        _nms_kernel,
        out_shape=jax.ShapeDtypeStruct((1, _NPAD), jnp.float32),
        scratch_shapes=[pltpu.VMEM((_B, _B), jnp.float32)],
    )(bsp, bst)


def kernel(boxes, scores):
    order = jnp.argsort(-scores)
    bs = jnp.take(boxes, order, axis=0)
    ss = jnp.take(scores, order)
    bsp = jnp.concatenate(
        [bs, jnp.zeros((_NPAD - _N, 4), jnp.float32)], axis=0)
    keepf = _run_nms(bsp, bsp.T)[0, :_N]
    keep = keepf > 0.5
    boxes_out = jnp.where(keep[:, None], bs, 0.0)
    scores_out = jnp.where(keep, ss, 0.0)
    return boxes_out, scores_out, keep
